# Initial kernel scaffold; baseline (speedup 1.0000x reference)
import jax
import jax.numpy as jnp
from jax.experimental import pallas as pl


def kernel(feature, edge_index, embedding, W_conv, W_mlp):
    raise NotImplementedError("write your pallas kernel here")


# speedup vs baseline: 8.3957x; 8.3957x over previous
import functools

import jax
import jax.numpy as jnp
from jax import lax
from jax.experimental import pallas as pl
from jax.experimental.pallas import tpu as pltpu
from jax.experimental.pallas import tpu_sc as plsc

N = 10000
E = 320000
D = 128

NC = 2
NS = 16

C = 128
CHUNKS = 160
EPAD = NS * CHUNKS * C
DH = D // 2

NPAD = 10240
RPT = NPAD // NS

HALF = NPAD // 2
HROWS = HALF + 1
RPT_H = HALF // NS

ZCH = 32
ICH = 32
WCH = 32
BROWS = 1024
FROWS = 80


def _sc_mesh():
    return plsc.VectorSubcoreMesh(
        core_axis_name="c", subcore_axis_name="s",
        num_cores=NC, num_subcores=NS)


def _prep_body(emb_ref, feat_ref, wc_ref, wm_ref, xw_ref, mlp_ref):
    xw_ref[...] = lax.dot_general(emb_ref[...], wc_ref[...],
                                  (((1,), (1,)), ((), ())),
                                  preferred_element_type=jnp.float32)
    mlp_ref[...] = lax.dot_general(feat_ref[...], wm_ref[...],
                                   (((1,), (1,)), ((), ())),
                                   preferred_element_type=jnp.float32)


def _prep(emb_pad, feat_pad, W_conv, W_mlp):
    grid = (NPAD // BROWS,)
    return pl.pallas_call(
        _prep_body,
        grid=grid,
        in_specs=[
            pl.BlockSpec((BROWS, D), lambda i: (i, 0)),
            pl.BlockSpec((BROWS, D), lambda i: (i, 0)),
            pl.BlockSpec((D, D), lambda i: (0, 0)),
            pl.BlockSpec((D, D), lambda i: (0, 0)),
        ],
        out_specs=[
            pl.BlockSpec((BROWS, D), lambda i: (i, 0)),
            pl.BlockSpec((BROWS, D), lambda i: (i, 0)),
        ],
        out_shape=[
            jax.ShapeDtypeStruct((NPAD, D), jnp.float32),
            jax.ShapeDtypeStruct((NPAD, D), jnp.float32),
        ],
    )(emb_pad, feat_pad, W_conv, W_mlp)


def _rsqrt16(x):
    xi = plsc.bitcast(x, jnp.int32)
    y = plsc.bitcast(jnp.int32(0x5F3759DF) - (xi >> 1), jnp.float32)
    for _ in range(3):
        y = y * (1.5 - 0.5 * x * y * y)
    return y


def _fused_body(src_hbm, dstd_hbm, xw_hbm, ones_hbm, zeros_hbm,
                acc_out, z_out, dinv_out,
                src_v, dstd_v, dstr_s, ones_v, buf, zw, zw0, zw1,
                dinv_v, dv, gsem, ssem, deg_sh, accum_sh):
    cid = lax.axis_index("c")
    sid = lax.axis_index("s")
    n0 = sid * RPT
    a0 = sid * RPT_H

    z16 = jnp.zeros((16,), jnp.float32)
    pltpu.sync_copy(zeros_hbm.at[pl.ds(n0, RPT)], deg_sh.at[pl.ds(n0, RPT)])
    pltpu.sync_copy(ones_hbm, ones_v)

    def zero_zw(i, carry):
        for q in range(DH // 16):
            zw0[i, pl.ds(q * 16, 16)] = z16
        return carry

    lax.fori_loop(0, ZCH, zero_zw, 0)

    def zero_acc(i, carry):
        pltpu.sync_copy(zw0, accum_sh.at[pl.ds(a0 + i * ZCH, ZCH)])
        return carry

    lax.fori_loop(0, RPT_H // ZCH, zero_acc, 0)

    def stage_idx(i, carry):
        pltpu.sync_copy(src_hbm.at[sid].at[pl.ds(i * ICH, ICH)],
                        src_v.at[pl.ds(i * ICH, ICH)])
        pltpu.sync_copy(dstd_hbm.at[sid].at[pl.ds(i * ICH, ICH)],
                        dstd_v.at[pl.ds(i * ICH, ICH)])
        return carry

    lax.fori_loop(0, CHUNKS // ICH, stage_idx, 0)

    plsc.subcore_barrier()

    DEGB = 8

    def deg_body(jj, carry):
        j0 = jj * DEGB
        descs = [
            pltpu.async_copy(ones_v, deg_sh.at[dstd_v.at[j0 + b]], gsem,
                             add=True)
            for b in range(DEGB)
        ]
        for dsc in descs:
            dsc.wait()
        return carry

    lax.fori_loop(0, CHUNKS // DEGB, deg_body, 0)
    plsc.subcore_barrier()

    pltpu.sync_copy(deg_sh.at[pl.ds(n0, RPT)], dv)

    def dinv_body(k, carry):
        x = dv[pl.ds(k * 16, 16)] + 1.0
        dinv_v[pl.ds(k * 16, 16)] = _rsqrt16(x)
        return carry

    lax.fori_loop(0, RPT // 16, dinv_body, 0)

    @pl.when(cid == 0)
    def _():
        pltpu.sync_copy(dinv_v, dinv_out.at[pl.ds(n0, RPT)])

    def zchunk_body(k, carry):
        r0 = n0 + k * ZCH
        pltpu.sync_copy(xw_hbm.at[pl.ds(r0, ZCH)], zw)

        def scale_body(r, carry2):
            bc = plsc.load_gather(dinv_v, [jnp.full((16,), k * ZCH + r,
                                                    jnp.int32)])
            for q in range(DH // 16):
                zw0[r, pl.ds(q * 16, 16)] = zw[r, pl.ds(q * 16, 16)] * bc
            for q in range(DH // 16):
                zw1[r, pl.ds(q * 16, 16)] = (
                    zw[r, pl.ds(DH + q * 16, 16)] * bc)
            return carry2

        lax.fori_loop(0, ZCH, scale_body, 0)
        pltpu.sync_copy(zw0, z_out.at[cid].at[0].at[pl.ds(r0, ZCH)])
        pltpu.sync_copy(zw1, z_out.at[cid].at[1].at[pl.ds(r0, ZCH)])
        return carry

    lax.fori_loop(0, RPT // ZCH, zchunk_body, 0)
    plsc.subcore_barrier()

    lo = cid * HALF
    for p in range(2):
        pltpu.async_copy(z_out.at[cid].at[p].at[src_v.at[0]], buf.at[0],
                         gsem)

        def msg_body(j, carry):
            par = lax.rem(j, 2)

            @pl.when(j > 0)
            def _():
                pltpu.make_async_copy(z_out.at[cid].at[p].at[pl.ds(0, C)],
                                      buf.at[0], ssem).wait()

            @pl.when(j < CHUNKS - 1)
            def _():
                pltpu.async_copy(z_out.at[cid].at[p].at[src_v.at[j + 1]],
                                 buf.at[lax.rem(j + 1, 2)], gsem)

            pltpu.make_async_copy(z_out.at[cid].at[p].at[pl.ds(0, C)],
                                  buf.at[0], gsem).wait()
            for q in range(C // 16):
                v = dstd_v[j, pl.ds(q * 16, 16)]
                inr = (v >= lo) & (v < lo + HALF)
                dstr_s[0, pl.ds(q * 16, 16)] = jnp.where(
                    inr, v - lo, jnp.int32(HALF))
            pltpu.async_copy(buf.at[par], accum_sh.at[dstr_s.at[0]],
                             ssem, add=True)
            return carry

        lax.fori_loop(0, CHUNKS, msg_body, 0)
        pltpu.make_async_copy(z_out.at[cid].at[p].at[pl.ds(0, C)],
                              buf.at[0], ssem).wait()
        plsc.subcore_barrier()

        def wb_body(i, carry):
            pltpu.sync_copy(
                accum_sh.at[pl.ds(a0 + i * WCH, WCH)],
                acc_out.at[cid].at[p].at[pl.ds(a0 + i * WCH, WCH)])
            return carry

        lax.fori_loop(0, RPT_H // WCH, wb_body, 0)
        if p == 0:
            def rz_body(i, carry):
                for q in range(DH // 16):
                    zw0[i, pl.ds(q * 16, 16)] = jnp.zeros((16,),
                                                          jnp.float32)
                return carry

            lax.fori_loop(0, ZCH, rz_body, 0)

            def rz_copy(i, carry):
                pltpu.sync_copy(zw0, accum_sh.at[pl.ds(a0 + i * ZCH, ZCH)])
                return carry

            lax.fori_loop(0, RPT_H // ZCH, rz_copy, 0)
            plsc.subcore_barrier()


def _fused(src_m, dst_m, xw_pad, ones_c, zeros_n):
    fn = pl.kernel(
        _fused_body,
        out_type=[
            jax.ShapeDtypeStruct((NC, 2, HALF, DH), jnp.float32),
            jax.ShapeDtypeStruct((NC, 2, NPAD, DH), jnp.float32),
            jax.ShapeDtypeStruct((NPAD,), jnp.float32),
        ],
        mesh=_sc_mesh(),
        compiler_params=pltpu.CompilerParams(needs_layout_passes=False,
                                             use_tc_tiling_on_sc=False),
        scratch_types=[
            pltpu.VMEM((CHUNKS, C), jnp.int32),
            pltpu.VMEM((CHUNKS, C), jnp.int32),
            pltpu.VMEM((1, C), jnp.int32),
            pltpu.VMEM((C,), jnp.float32),
            pltpu.VMEM((2, C, DH), jnp.float32),
            pltpu.VMEM((ZCH, D), jnp.float32),
            pltpu.VMEM((ZCH, DH), jnp.float32),
            pltpu.VMEM((ZCH, DH), jnp.float32),
            pltpu.VMEM((RPT,), jnp.float32),
            pltpu.VMEM((RPT,), jnp.float32),
            pltpu.SemaphoreType.DMA,
            pltpu.SemaphoreType.DMA,
            pltpu.VMEM_SHARED((NPAD,), jnp.float32),
            pltpu.VMEM_SHARED((HROWS, DH), jnp.float32),
        ],
    )
    return fn(src_m, dst_m, xw_pad, ones_c, zeros_n)


def _final_body(acc0_ref, acc1_ref, xw_ref, mlp_ref, dinv_ref, out_ref):
    dinv = dinv_ref[...]
    acc = jnp.concatenate([acc0_ref[0, 0], acc1_ref[0, 0]], axis=-1)
    s = (acc * dinv + xw_ref[...] * (dinv * dinv)) + mlp_ref[...]
    out_ref[...] = jnp.maximum(s, 0.0)


def _final(acc2, xw_pad, mlp_pad, dinv2):
    grid = (N // FROWS,)
    return pl.pallas_call(
        _final_body,
        grid=grid,
        in_specs=[
            pl.BlockSpec((1, 1, FROWS, DH),
                         lambda i: ((i * FROWS) // HALF, 0,
                                    (i * FROWS % HALF) // FROWS, 0)),
            pl.BlockSpec((1, 1, FROWS, DH),
                         lambda i: ((i * FROWS) // HALF, 1,
                                    (i * FROWS % HALF) // FROWS, 0)),
            pl.BlockSpec((FROWS, D), lambda i: (i, 0)),
            pl.BlockSpec((FROWS, D), lambda i: (i, 0)),
            pl.BlockSpec((FROWS, 1), lambda i: (i, 0)),
        ],
        out_specs=pl.BlockSpec((FROWS, D), lambda i: (i, 0)),
        out_shape=jax.ShapeDtypeStruct((N, D), jnp.float32),
    )(acc2, acc2, xw_pad, mlp_pad, dinv2)


def kernel(feature, edge_index, embedding, W_conv, W_mlp):
    npad_e = EPAD - E
    src_p = jnp.concatenate(
        [edge_index[0].astype(jnp.int32), jnp.zeros((npad_e,), jnp.int32)])
    dst_p = jnp.concatenate(
        [edge_index[1].astype(jnp.int32),
         jnp.full((npad_e,), NPAD - 1, jnp.int32)])
    src_m = src_p.reshape(NS, CHUNKS, C)
    dst_m = dst_p.reshape(NS, CHUNKS, C)
    emb_pad = jnp.pad(embedding, ((0, NPAD - N), (0, 0)))
    feat_pad = jnp.pad(feature, ((0, NPAD - N), (0, 0)))

    ones_c = jnp.ones((C,), jnp.float32)
    zeros_n = jnp.zeros((NPAD,), jnp.float32)
    xw_pad, mlp_pad = _prep(emb_pad, feat_pad, W_conv, W_mlp)
    acc2, _z, dinv = _fused(src_m, dst_m, xw_pad, ones_c, zeros_n)
    return _final(acc2, xw_pad, mlp_pad, dinv.reshape(NPAD, 1))
